# baseline (speedup 1.0000x reference)
import functools

import jax
import jax.numpy as jnp
from jax import lax
from jax.experimental import pallas as pl
from jax.experimental.pallas import tpu as pltpu
from jax.experimental.pallas import tpu_sc as plsc

F32 = jnp.float32
BF16 = jnp.bfloat16
BN = 1000
BE = 3200
KC = 200


def _ln(h, g, b):
    mu = jnp.mean(h, axis=-1, keepdims=True)
    var = jnp.mean((h - mu) ** 2, axis=-1, keepdims=True)
    return (h - mu) * lax.rsqrt(var + 1e-5) * g + b


def _silu(x):
    return x * lax.logistic(x)


def _bdot(a, w):
    return jnp.dot(a.astype(BF16), w.astype(BF16), preferred_element_type=F32)


def _ab_body(h_ref, wd_ref, ws_ref, b1_ref, a_ref, b_ref):
    h = h_ref[...]
    a_ref[...] = _bdot(h, wd_ref[...]) + b1_ref[...]
    b_ref[...] = _bdot(h, ws_ref[...])


def _ab(h, wd, ws, b1):
    N, C = h.shape
    return pl.pallas_call(
        _ab_body,
        grid=(N // BN,),
        in_specs=[
            pl.BlockSpec((BN, C), lambda i: (i, 0)),
            pl.BlockSpec((C, C), lambda i: (0, 0)),
            pl.BlockSpec((C, C), lambda i: (0, 0)),
            pl.BlockSpec((1, C), lambda i: (0, 0)),
        ],
        out_specs=[pl.BlockSpec((BN, C), lambda i: (i, 0)),
                   pl.BlockSpec((BN, C), lambda i: (i, 0))],
        out_shape=[jax.ShapeDtypeStruct((N, C), F32)] * 2,
    )(h, wd, ws, b1.reshape(1, C))


def _epre_body(e_ref, w_ref, o_ref):
    o_ref[...] = _bdot(e_ref[...], w_ref[...]).astype(BF16)


def _epre(e, w):
    E, D = e.shape
    C = w.shape[1]
    return pl.pallas_call(
        _epre_body,
        grid=(E // BE,),
        in_specs=[pl.BlockSpec((BE, D), lambda i: (i, 0)),
                  pl.BlockSpec((D, C), lambda i: (0, 0))],
        out_specs=pl.BlockSpec((BE, C), lambda i: (i, 0)),
        out_shape=jax.ShapeDtypeStruct((E, C), BF16),
    )(e, w)


def _make_edge_body(with_res, with_next):
    def body(*refs):
        refs = list(refs)
        ga_ref, gb_ref, ep_ref = refs[:3]
        i = 3
        ev_ref = None
        if with_res:
            ev_ref = refs[i]
            i += 1
        w2_ref, b2_ref, g_ref, bl_ref = refs[i:i + 4]
        i += 4
        wn_ref = None
        if with_next:
            wn_ref = refs[i]
            i += 1
        o_ref = refs[i]
        i += 1
        hid = _silu(ga_ref[...] + gb_ref[...] + ep_ref[...].astype(F32))
        out = _bdot(hid, w2_ref[...]) + b2_ref[...]
        e_new = _ln(out, g_ref[...], bl_ref[...])
        if with_res:
            e_new = e_new + ev_ref[...]
        o_ref[...] = e_new
        if with_next:
            refs[i][...] = _bdot(e_new, wn_ref[...]).astype(BF16)
    return body


def _edge(ga, gb, ep, em, e_prev, w1e_next):
    E, C = ga.shape
    blk = lambda: pl.BlockSpec((BE, C), lambda i: (i, 0))
    cc = lambda: pl.BlockSpec((C, C), lambda i: (0, 0))
    rc = lambda: pl.BlockSpec((1, C), lambda i: (0, 0))
    with_res = e_prev is not None
    with_next = w1e_next is not None
    args = [ga, gb, ep]
    in_specs = [blk(), blk(), blk()]
    if with_res:
        args.append(e_prev)
        in_specs.append(blk())
    args += [em["W2"], em["b2"].reshape(1, C), em["ln_g"].reshape(1, C),
             em["ln_b"].reshape(1, C)]
    in_specs += [cc(), rc(), rc(), rc()]
    if with_next:
        args.append(w1e_next)
        in_specs.append(cc())
    out_specs = [blk()]
    out_shape = [jax.ShapeDtypeStruct((E, C), F32)]
    if with_next:
        out_specs.append(blk())
        out_shape.append(jax.ShapeDtypeStruct((E, C), BF16))
    res = pl.pallas_call(
        _make_edge_body(with_res, with_next),
        grid=(E // BE,),
        in_specs=in_specs,
        out_specs=out_specs,
        out_shape=out_shape,
    )(*args)
    return res if with_next else (res[0], None)


def _node_body(h_ref, p0_ref, p1_ref, p2_ref, p3_ref, wh_ref, wa_ref, b1_ref,
               w2_ref, b2_ref, g_ref, bl_ref, ho_ref):
    h = h_ref[...]
    agg = (p0_ref[...] + p1_ref[...]) + (p2_ref[...] + p3_ref[...])
    hid = _silu(_bdot(h, wh_ref[...]) + _bdot(agg, wa_ref[...]) + b1_ref[...])
    out = _bdot(hid, w2_ref[...]) + b2_ref[...]
    ho_ref[...] = _ln(out, g_ref[...], bl_ref[...]) + h


def _node_body_ab(h_ref, p0_ref, p1_ref, p2_ref, p3_ref, wh_ref, wa_ref,
                  b1_ref, w2_ref, b2_ref, g_ref, bl_ref, wdn_ref, wsn_ref,
                  b1n_ref, ho_ref, a_ref, b_ref):
    h = h_ref[...]
    agg = (p0_ref[...] + p1_ref[...]) + (p2_ref[...] + p3_ref[...])
    hid = _silu(_bdot(h, wh_ref[...]) + _bdot(agg, wa_ref[...]) + b1_ref[...])
    out = _bdot(hid, w2_ref[...]) + b2_ref[...]
    hn = _ln(out, g_ref[...], bl_ref[...]) + h
    ho_ref[...] = hn
    a_ref[...] = _bdot(hn, wdn_ref[...]) + b1n_ref[...]
    b_ref[...] = _bdot(hn, wsn_ref[...])


def _node(h, ps, nm, next_em):
    N, C = h.shape
    blk = lambda: pl.BlockSpec((BN, C), lambda i: (i, 0))
    cc = lambda: pl.BlockSpec((C, C), lambda i: (0, 0))
    rc = lambda: pl.BlockSpec((1, C), lambda i: (0, 0))
    W1 = nm["W1"]
    wargs = (W1[:C], W1[C:], nm["b1"].reshape(1, C), nm["W2"],
             nm["b2"].reshape(1, C), nm["ln_g"].reshape(1, C),
             nm["ln_b"].reshape(1, C))
    if next_em is None:
        return pl.pallas_call(
            _node_body,
            grid=(N // BN,),
            in_specs=[blk()] * 5 + [cc(), cc(), rc(), cc(), rc(),
                      rc(), rc()],
            out_specs=blk(),
            out_shape=jax.ShapeDtypeStruct((N, C), F32),
        )(h, *ps, *wargs)
    nW1 = next_em["W1"]
    return pl.pallas_call(
        _node_body_ab,
        grid=(N // BN,),
        in_specs=[blk()] * 5 + [cc(), cc(), rc(), cc(), rc(),
                  rc(), rc(), cc(), cc(), rc()],
        out_specs=[blk(), blk(), blk()],
        out_shape=[jax.ShapeDtypeStruct((N, C), F32)] * 3,
    )(h, *ps, *wargs, nW1[:C], nW1[C:2 * C], next_em["b1"].reshape(1, C))


def _make_sc_fns(N, C, E):
    info = plsc.get_sparse_core_info()
    ncore, nsub = info.num_cores, info.num_subcores
    nw = ncore * nsub
    epw = E // nw
    assert E % nw == 0 and epw % KC == 0
    nchunks = epw // KC
    npairs = nchunks // 2
    tail = nchunks % 2
    rfull = -(-N // (nsub * 8)) * 8
    rlast = N - rfull * (nsub - 1)
    assert rlast > 0 and rlast % 8 == 0
    mesh = plsc.VectorSubcoreMesh(core_axis_name="c", subcore_axis_name="s")

    @functools.partial(
        pl.kernel,
        out_type=(jax.ShapeDtypeStruct((E, C), F32),
                  jax.ShapeDtypeStruct((E, C), F32)),
        mesh=mesh,
        scratch_types=[
            pltpu.VMEM((KC,), jnp.int32),
            pltpu.VMEM((KC,), jnp.int32),
            pltpu.VMEM((KC, C), F32),
            pltpu.VMEM((KC, C), F32),
            pltpu.SemaphoreType.DMA,
            pltpu.SemaphoreType.DMA,
        ],
    )
    def gather(a_hbm, b_hbm, dst_hbm, src_hbm, oa_hbm, ob_hbm,
               di, si, ra, rb, s1, s2):
        wid = lax.axis_index("c") * nsub + lax.axis_index("s")
        base = wid * epw

        @pl.loop(0, nchunks)
        def _(k):
            off = base + k * KC
            pltpu.sync_copy(dst_hbm.at[pl.ds(off, KC)], di)
            pltpu.sync_copy(src_hbm.at[pl.ds(off, KC)], si)
            cpa = pltpu.async_copy(a_hbm.at[di], ra, s1)
            cpb = pltpu.async_copy(b_hbm.at[si], rb, s2)
            cpa.wait()
            cpb.wait()
            pltpu.sync_copy(ra, oa_hbm.at[pl.ds(off, KC)])
            pltpu.sync_copy(rb, ob_hbm.at[pl.ds(off, KC)])

    @functools.partial(
        pl.kernel,
        out_type=jax.ShapeDtypeStruct((ncore, N, C), F32),
        mesh=mesh,
        scratch_types=[
            pltpu.VMEM_SHARED((N, C), F32),
            pltpu.VMEM((KC,), jnp.int32),
            pltpu.VMEM((KC, C), F32),
            pltpu.SemaphoreType.DMA,
        ],
    )
    def scatter(e_hbm, dst_hbm, zero_hbm, o_hbm, acc, di0, r0, se0):
        c = lax.axis_index("c")
        s = lax.axis_index("s")
        base = (c * nsub + s) * epw
        row0 = s * rfull

        @pl.when(s < nsub - 1)
        def _():
            pltpu.sync_copy(zero_hbm.at[pl.ds(row0, rfull)],
                            acc.at[pl.ds(row0, rfull)])

        @pl.when(s == nsub - 1)
        def _():
            pltpu.sync_copy(zero_hbm.at[pl.ds(row0, rlast)],
                            acc.at[pl.ds(row0, rlast)])

        plsc.subcore_barrier()

        @pl.loop(0, nchunks)
        def _(k):
            off = base + k * KC
            pltpu.sync_copy(dst_hbm.at[pl.ds(off, KC)], di0)
            pltpu.sync_copy(e_hbm.at[pl.ds(off, KC)], r0)
            pltpu.sync_copy(r0, acc.at[di0], add=True)

        plsc.subcore_barrier()

        @pl.when(s < nsub - 1)
        def _():
            pltpu.sync_copy(acc.at[pl.ds(row0, rfull)],
                            o_hbm.at[c, pl.ds(row0, rfull)])

        @pl.when(s == nsub - 1)
        def _():
            pltpu.sync_copy(acc.at[pl.ds(row0, rlast)],
                            o_hbm.at[c, pl.ds(row0, rlast)])

    return gather, scatter


def kernel(x, edge_attr, edge_index, params, batch_size, shard_shapes):
    N, C = x.shape
    E = edge_index.shape[1]
    src = edge_index[0]
    dst = edge_index[1]
    assert N % BN == 0

    step = 32 * KC
    cut = (E // 2 // (step * 2)) * step * 2
    while cut % BE or (E - cut) % BE or (E - cut) % step:
        cut -= step
    bounds = [(0, cut), (cut, E)]
    dsts = [lax.slice(dst, (lo,), (hi,)) for lo, hi in bounds]
    srcs = [lax.slice(src, (lo,), (hi,)) for lo, hi in bounds]
    sc_fns = {hi - lo: _make_sc_fns(N, C, hi - lo) for lo, hi in bounds}
    zeros_nc = jnp.zeros((N, C), F32)

    h = x
    es = [lax.slice(edge_attr, (lo, 0), (hi, edge_attr.shape[1]))
          for lo, hi in bounds]
    L = len(params)
    A = B = None
    epres = [None, None]
    for l, p in enumerate(params):
        em = p["edge_mlp"]
        W1 = em["W1"]
        if l == 0:
            A, B = _ab(h, W1[:C], W1[C:2 * C], em["b1"])
            epres = [_epre(eh, W1[2 * C:]) for eh in es]
        w1e_next = (params[l + 1]["edge_mlp"]["W1"][2 * C:]
                    if l + 1 < L else None)
        gs = []
        for i, (lo, hi) in enumerate(bounds):
            gather, _ = sc_fns[hi - lo]
            gs.append(gather(A, B, dsts[i], srcs[i]))
        e_news = [None, None]
        partials = []
        for i, (lo, hi) in enumerate(bounds):
            _, scatter = sc_fns[hi - lo]
            e_news[i], epres[i] = _edge(gs[i][0], gs[i][1], epres[i], em,
                                        es[i] if l > 0 else None, w1e_next)
            partials.append(scatter(e_news[i], dsts[i], zeros_nc))
        ps = [partials[0][0], partials[0][1], partials[1][0], partials[1][1]]
        if l + 1 < L:
            h, A, B = _node(h, ps, p["node_mlp"], params[l + 1]["edge_mlp"])
        else:
            h = _node(h, ps, p["node_mlp"], None)
        es = e_news
    return h


# speedup vs baseline: 1.0196x; 1.0196x over previous
import functools

import jax
import jax.numpy as jnp
from jax import lax
from jax.experimental import pallas as pl
from jax.experimental.pallas import tpu as pltpu
from jax.experimental.pallas import tpu_sc as plsc

F32 = jnp.float32
BF16 = jnp.bfloat16
BN = 1000
BE = 3200
KC = 200


def _ln(h, g, b):
    mu = jnp.mean(h, axis=-1, keepdims=True)
    var = jnp.mean((h - mu) ** 2, axis=-1, keepdims=True)
    return (h - mu) * lax.rsqrt(var + 1e-5) * g + b


def _silu(x):
    return x * lax.logistic(x)


def _bdot(a, w):
    return jnp.dot(a.astype(BF16), w.astype(BF16), preferred_element_type=F32)


def _ab_body(h_ref, wd_ref, ws_ref, b1_ref, a_ref, b_ref):
    h = h_ref[...]
    a_ref[...] = _bdot(h, wd_ref[...]) + b1_ref[...]
    b_ref[...] = _bdot(h, ws_ref[...])


def _ab(h, wd, ws, b1):
    N, C = h.shape
    return pl.pallas_call(
        _ab_body,
        grid=(N // BN,),
        in_specs=[
            pl.BlockSpec((BN, C), lambda i: (i, 0)),
            pl.BlockSpec((C, C), lambda i: (0, 0)),
            pl.BlockSpec((C, C), lambda i: (0, 0)),
            pl.BlockSpec((1, C), lambda i: (0, 0)),
        ],
        out_specs=[pl.BlockSpec((BN, C), lambda i: (i, 0)),
                   pl.BlockSpec((BN, C), lambda i: (i, 0))],
        out_shape=[jax.ShapeDtypeStruct((N, C), F32)] * 2,
    )(h, wd, ws, b1.reshape(1, C))


def _epre_body(e_ref, w_ref, o_ref):
    o_ref[...] = _bdot(e_ref[...], w_ref[...]).astype(BF16)


def _epre(e, w):
    E, D = e.shape
    C = w.shape[1]
    return pl.pallas_call(
        _epre_body,
        grid=(E // BE,),
        in_specs=[pl.BlockSpec((BE, D), lambda i: (i, 0)),
                  pl.BlockSpec((D, C), lambda i: (0, 0))],
        out_specs=pl.BlockSpec((BE, C), lambda i: (i, 0)),
        out_shape=jax.ShapeDtypeStruct((E, C), BF16),
    )(e, w)


def _make_edge_body(with_res, with_next):
    def body(*refs):
        refs = list(refs)
        ga_ref, gb_ref, ep_ref = refs[:3]
        i = 3
        ev_ref = None
        if with_res:
            ev_ref = refs[i]
            i += 1
        w2_ref, b2_ref, g_ref, bl_ref = refs[i:i + 4]
        i += 4
        wn_ref = None
        if with_next:
            wn_ref = refs[i]
            i += 1
        o_ref = refs[i]
        i += 1
        hid = _silu(ga_ref[...] + gb_ref[...] + ep_ref[...].astype(F32))
        out = _bdot(hid, w2_ref[...]) + b2_ref[...]
        e_new = _ln(out, g_ref[...], bl_ref[...])
        if with_res:
            e_new = e_new + ev_ref[...]
        o_ref[...] = e_new
        if with_next:
            refs[i][...] = _bdot(e_new, wn_ref[...]).astype(BF16)
    return body


def _edge(ga, gb, ep, em, e_prev, w1e_next):
    E, C = ga.shape
    blk = lambda: pl.BlockSpec((BE, C), lambda i: (i, 0))
    cc = lambda: pl.BlockSpec((C, C), lambda i: (0, 0))
    rc = lambda: pl.BlockSpec((1, C), lambda i: (0, 0))
    with_res = e_prev is not None
    with_next = w1e_next is not None
    args = [ga, gb, ep]
    in_specs = [blk(), blk(), blk()]
    if with_res:
        args.append(e_prev)
        in_specs.append(blk())
    args += [em["W2"], em["b2"].reshape(1, C), em["ln_g"].reshape(1, C),
             em["ln_b"].reshape(1, C)]
    in_specs += [cc(), rc(), rc(), rc()]
    if with_next:
        args.append(w1e_next)
        in_specs.append(cc())
    out_specs = [blk()]
    out_shape = [jax.ShapeDtypeStruct((E, C), F32)]
    if with_next:
        out_specs.append(blk())
        out_shape.append(jax.ShapeDtypeStruct((E, C), BF16))
    res = pl.pallas_call(
        _make_edge_body(with_res, with_next),
        grid=(E // BE,),
        in_specs=in_specs,
        out_specs=out_specs,
        out_shape=out_shape,
    )(*args)
    return res if with_next else (res[0], None)


def _node_body(h_ref, p0_ref, p1_ref, p2_ref, p3_ref, wh_ref, wa_ref, b1_ref,
               w2_ref, b2_ref, g_ref, bl_ref, ho_ref):
    h = h_ref[...]
    agg = (p0_ref[...] + p1_ref[...]) + (p2_ref[...] + p3_ref[...])
    hid = _silu(_bdot(h, wh_ref[...]) + _bdot(agg, wa_ref[...]) + b1_ref[...])
    out = _bdot(hid, w2_ref[...]) + b2_ref[...]
    ho_ref[...] = _ln(out, g_ref[...], bl_ref[...]) + h


def _node_body_ab(h_ref, p0_ref, p1_ref, p2_ref, p3_ref, wh_ref, wa_ref,
                  b1_ref, w2_ref, b2_ref, g_ref, bl_ref, wdn_ref, wsn_ref,
                  b1n_ref, ho_ref, a_ref, b_ref):
    h = h_ref[...]
    agg = (p0_ref[...] + p1_ref[...]) + (p2_ref[...] + p3_ref[...])
    hid = _silu(_bdot(h, wh_ref[...]) + _bdot(agg, wa_ref[...]) + b1_ref[...])
    out = _bdot(hid, w2_ref[...]) + b2_ref[...]
    hn = _ln(out, g_ref[...], bl_ref[...]) + h
    ho_ref[...] = hn
    a_ref[...] = _bdot(hn, wdn_ref[...]) + b1n_ref[...]
    b_ref[...] = _bdot(hn, wsn_ref[...])


def _node(h, ps, nm, next_em):
    N, C = h.shape
    blk = lambda: pl.BlockSpec((BN, C), lambda i: (i, 0))
    cc = lambda: pl.BlockSpec((C, C), lambda i: (0, 0))
    rc = lambda: pl.BlockSpec((1, C), lambda i: (0, 0))
    W1 = nm["W1"]
    wargs = (W1[:C], W1[C:], nm["b1"].reshape(1, C), nm["W2"],
             nm["b2"].reshape(1, C), nm["ln_g"].reshape(1, C),
             nm["ln_b"].reshape(1, C))
    if next_em is None:
        return pl.pallas_call(
            _node_body,
            grid=(N // BN,),
            in_specs=[blk()] * 5 + [cc(), cc(), rc(), cc(), rc(),
                      rc(), rc()],
            out_specs=blk(),
            out_shape=jax.ShapeDtypeStruct((N, C), F32),
        )(h, *ps, *wargs)
    nW1 = next_em["W1"]
    return pl.pallas_call(
        _node_body_ab,
        grid=(N // BN,),
        in_specs=[blk()] * 5 + [cc(), cc(), rc(), cc(), rc(),
                  rc(), rc(), cc(), cc(), rc()],
        out_specs=[blk(), blk(), blk()],
        out_shape=[jax.ShapeDtypeStruct((N, C), F32)] * 3,
    )(h, *ps, *wargs, nW1[:C], nW1[C:2 * C], next_em["b1"].reshape(1, C))


def _make_sc_fns(N, C, E):
    info = plsc.get_sparse_core_info()
    ncore, nsub = info.num_cores, info.num_subcores
    nw = ncore * nsub
    epw = E // nw
    assert E % nw == 0 and epw % KC == 0
    nchunks = epw // KC
    kc_g = 400 if epw % 400 == 0 else KC
    ng = epw // kc_g
    rfull = -(-N // (nsub * 8)) * 8
    rlast = N - rfull * (nsub - 1)
    assert rlast > 0 and rlast % 8 == 0
    mesh = plsc.VectorSubcoreMesh(core_axis_name="c", subcore_axis_name="s")

    @functools.partial(
        pl.kernel,
        out_type=(jax.ShapeDtypeStruct((E, C), F32),
                  jax.ShapeDtypeStruct((E, C), F32)),
        mesh=mesh,
        scratch_types=[
            pltpu.VMEM((kc_g,), jnp.int32),
            pltpu.VMEM((kc_g,), jnp.int32),
            pltpu.VMEM((kc_g, C), F32),
            pltpu.VMEM((kc_g, C), F32),
            pltpu.SemaphoreType.DMA,
            pltpu.SemaphoreType.DMA,
        ],
    )
    def gather(a_hbm, b_hbm, dst_hbm, src_hbm, oa_hbm, ob_hbm,
               di, si, ra, rb, s1, s2):
        wid = lax.axis_index("c") * nsub + lax.axis_index("s")
        base = wid * epw

        @pl.loop(0, ng)
        def _(k):
            off = base + k * kc_g
            pltpu.sync_copy(dst_hbm.at[pl.ds(off, kc_g)], di)
            pltpu.sync_copy(src_hbm.at[pl.ds(off, kc_g)], si)
            cpa = pltpu.async_copy(a_hbm.at[di], ra, s1)
            cpb = pltpu.async_copy(b_hbm.at[si], rb, s2)
            cpa.wait()
            cpb.wait()
            pltpu.sync_copy(ra, oa_hbm.at[pl.ds(off, kc_g)])
            pltpu.sync_copy(rb, ob_hbm.at[pl.ds(off, kc_g)])

    @functools.partial(
        pl.kernel,
        out_type=jax.ShapeDtypeStruct((ncore, N, C), F32),
        mesh=mesh,
        scratch_types=[
            pltpu.VMEM_SHARED((N, C), F32),
            pltpu.VMEM((KC,), jnp.int32),
            pltpu.VMEM((KC, C), F32),
            pltpu.SemaphoreType.DMA,
        ],
    )
    def scatter(e_hbm, dst_hbm, zero_hbm, o_hbm, acc, di0, r0, se0):
        c = lax.axis_index("c")
        s = lax.axis_index("s")
        base = (c * nsub + s) * epw
        row0 = s * rfull

        @pl.when(s < nsub - 1)
        def _():
            pltpu.sync_copy(zero_hbm.at[pl.ds(row0, rfull)],
                            acc.at[pl.ds(row0, rfull)])

        @pl.when(s == nsub - 1)
        def _():
            pltpu.sync_copy(zero_hbm.at[pl.ds(row0, rlast)],
                            acc.at[pl.ds(row0, rlast)])

        plsc.subcore_barrier()

        @pl.loop(0, nchunks)
        def _(k):
            off = base + k * KC
            pltpu.sync_copy(dst_hbm.at[pl.ds(off, KC)], di0)
            pltpu.sync_copy(e_hbm.at[pl.ds(off, KC)], r0)
            pltpu.sync_copy(r0, acc.at[di0], add=True)

        plsc.subcore_barrier()

        @pl.when(s < nsub - 1)
        def _():
            pltpu.sync_copy(acc.at[pl.ds(row0, rfull)],
                            o_hbm.at[c, pl.ds(row0, rfull)])

        @pl.when(s == nsub - 1)
        def _():
            pltpu.sync_copy(acc.at[pl.ds(row0, rlast)],
                            o_hbm.at[c, pl.ds(row0, rlast)])

    return gather, scatter


def kernel(x, edge_attr, edge_index, params, batch_size, shard_shapes):
    N, C = x.shape
    E = edge_index.shape[1]
    src = edge_index[0]
    dst = edge_index[1]
    assert N % BN == 0

    step = 32 * KC
    cut = (E // 2 // (step * 2)) * step * 2
    while cut % BE or (E - cut) % BE or (E - cut) % step:
        cut -= step
    bounds = [(0, cut), (cut, E)]
    dsts = [lax.slice(dst, (lo,), (hi,)) for lo, hi in bounds]
    srcs = [lax.slice(src, (lo,), (hi,)) for lo, hi in bounds]
    sc_fns = {hi - lo: _make_sc_fns(N, C, hi - lo) for lo, hi in bounds}
    zeros_nc = jnp.zeros((N, C), F32)

    h = x
    es = [lax.slice(edge_attr, (lo, 0), (hi, edge_attr.shape[1]))
          for lo, hi in bounds]
    L = len(params)
    A = B = None
    epres = [None, None]
    for l, p in enumerate(params):
        em = p["edge_mlp"]
        W1 = em["W1"]
        if l == 0:
            A, B = _ab(h, W1[:C], W1[C:2 * C], em["b1"])
            epres = [_epre(eh, W1[2 * C:]) for eh in es]
        w1e_next = (params[l + 1]["edge_mlp"]["W1"][2 * C:]
                    if l + 1 < L else None)
        gs = []
        for i, (lo, hi) in enumerate(bounds):
            gather, _ = sc_fns[hi - lo]
            gs.append(gather(A, B, dsts[i], srcs[i]))
        e_news = [None, None]
        partials = []
        for i, (lo, hi) in enumerate(bounds):
            _, scatter = sc_fns[hi - lo]
            e_news[i], epres[i] = _edge(gs[i][0], gs[i][1], epres[i], em,
                                        es[i] if l > 0 else None, w1e_next)
            partials.append(scatter(e_news[i], dsts[i], zeros_nc))
        ps = [partials[0][0], partials[0][1], partials[1][0], partials[1][1]]
        if l + 1 < L:
            h, A, B = _node(h, ps, p["node_mlp"], params[l + 1]["edge_mlp"])
        else:
            h = _node(h, ps, p["node_mlp"], None)
        es = e_news
    return h
